# Initial kernel scaffold; baseline (speedup 1.0000x reference)
import jax
import jax.numpy as jnp
from jax.experimental import pallas as pl


def kernel(tone_out, idx):
    raise NotImplementedError("write your pallas kernel here")


# speedup vs baseline: 77.2095x; 77.2095x over previous
import jax
import jax.numpy as jnp
from jax.experimental import pallas as pl

_B, _C, _P, _T = 8, 64, 12, 1024
_O = 6
_G = 8


def _unpool_body(tone_ref, idx_ref, out_ref):
    ix = idx_ref[...][:, None, :, :]
    tv = tone_ref[...][:, None, :, :]
    o_ids = jax.lax.broadcasted_iota(jnp.int32, (_G, _O, _P, _T), 1)
    out_ref[...] = jnp.where(ix == o_ids, tv, 0.0)


def kernel(tone_out, idx):
    b, c, p, t = tone_out.shape
    n = b * c
    tone2 = tone_out.reshape(n, p, t)
    idx2 = idx.reshape(n, p, t)
    out = pl.pallas_call(
        _unpool_body,
        grid=(n // _G,),
        in_specs=[
            pl.BlockSpec((_G, p, t), lambda i: (i, 0, 0)),
            pl.BlockSpec((_G, p, t), lambda i: (i, 0, 0)),
        ],
        out_specs=pl.BlockSpec((_G, _O, p, t), lambda i: (i, 0, 0, 0)),
        out_shape=jax.ShapeDtypeStruct((n, _O, p, t), tone_out.dtype),
    )(tone2, idx2)
    return out.reshape(b, c, _O * p, t)


# speedup vs baseline: 91.6888x; 1.1875x over previous
import jax
import jax.numpy as jnp
from jax.experimental import pallas as pl

_B, _C, _P, _T = 8, 64, 12, 1024
_O = 6
_G = 16


def _unpool_body(tone_ref, idx_ref, out_ref):
    ix = idx_ref[...][:, None]
    tv = tone_ref[...][:, None]
    o_ids = jax.lax.broadcasted_iota(
        jnp.int32, (_G, _O, _P, 8, 128), 1)
    out_ref[...] = jnp.where(ix == o_ids, tv, 0.0)


def kernel(tone_out, idx):
    b, c, p, t = tone_out.shape
    n = b * c
    tone2 = tone_out.reshape(n, p, 8, 128)
    idx2 = idx.reshape(n, p, 8, 128)
    out = pl.pallas_call(
        _unpool_body,
        grid=(n // _G,),
        in_specs=[
            pl.BlockSpec((_G, p, 8, 128), lambda i: (i, 0, 0, 0)),
            pl.BlockSpec((_G, p, 8, 128), lambda i: (i, 0, 0, 0)),
        ],
        out_specs=pl.BlockSpec((_G, _O, p, 8, 128), lambda i: (i, 0, 0, 0, 0)),
        out_shape=jax.ShapeDtypeStruct((n, _O, p, 8, 128), tone_out.dtype),
    )(tone2, idx2)
    return out.reshape(b, c, _O * p, t)


# speedup vs baseline: 435.8598x; 4.7537x over previous
import jax
import jax.numpy as jnp
from jax.experimental import pallas as pl

_B, _C, _P, _T = 8, 64, 12, 1024
_O = 6
_G = 16


def _unpool_body(tone_ref, idx_ref, out_ref):
    tv = tone_ref[...]
    ix = idx_ref[...]
    for o in range(_O):
        out_ref[:, o * _P:(o + 1) * _P, :] = jnp.where(ix == o, tv, 0.0)


def kernel(tone_out, idx):
    b, c, p, t = tone_out.shape
    n = b * c
    tone2 = tone_out.reshape(n, p, t)
    idx2 = idx.reshape(n, p, t)
    out = pl.pallas_call(
        _unpool_body,
        grid=(n // _G,),
        in_specs=[
            pl.BlockSpec((_G, p, t), lambda i: (i, 0, 0)),
            pl.BlockSpec((_G, p, t), lambda i: (i, 0, 0)),
        ],
        out_specs=pl.BlockSpec((_G, _O * p, t), lambda i: (i, 0, 0)),
        out_shape=jax.ShapeDtypeStruct((n, _O * p, t), tone_out.dtype),
    )(tone2, idx2)
    return out.reshape(b, c, _O * p, t)
